# Initial kernel scaffold; baseline (speedup 1.0000x reference)
import jax
import jax.numpy as jnp
from jax.experimental import pallas as pl


def kernel(x, edge_index, edge_attr, Wq, Wk, Wv, We, Wo, w_q_norm, w_k_norm):
    raise NotImplementedError("write your pallas kernel here")


# speedup vs baseline: 9.2123x; 9.2123x over previous
import functools

import jax
import jax.numpy as jnp
import numpy as np
from jax import lax
from jax.experimental import pallas as pl
from jax.experimental.pallas import tpu as pltpu
from jax.experimental.pallas import tpu_sc as plsc

N = 10000
E = 320000
D = 128
H = 8
C = 16

NC = 2
NS = 16
NW = NC * NS
EPW = E // NW
EB = 40
GB = 8
ZGRP = 16
NG = EB // GB
NCHUNK = EPW // EB
NPAD = 10240
DROWS = NPAD // 8
TROWS = NPAD + DROWS
NROWS_PER_SUB = NPAD // NS
DROWS_PER_SUB = DROWS // NS
ZROWS_PER_SUB = TROWS // NS
DRAIN_ROWS = 128

QK_SCALE = 1.0 / np.sqrt(float(C))


def _qkv_body(x_ref, w_ref, s_ref, aux_ref, q_ref, kv_ref):
    y = jnp.dot(x_ref[...], w_ref[...], preferred_element_type=jnp.float32)
    g = jnp.dot(y * y, s_ref[...], preferred_element_type=jnp.float32)
    z = y * lax.rsqrt(g + aux_ref[0]) * aux_ref[1]
    q_ref[...] = z[:, :D]
    kv_ref[...] = z[:, D:]


def _qkv(x, Wqkv, S, aux):
    blk = 1000
    return pl.pallas_call(
        _qkv_body,
        grid=(N // blk,),
        in_specs=[
            pl.BlockSpec((blk, D), lambda i: (i, 0)),
            pl.BlockSpec((D, 3 * D), lambda i: (0, 0)),
            pl.BlockSpec((3 * D, 3 * D), lambda i: (0, 0)),
            pl.BlockSpec((2, 3 * D), lambda i: (0, 0)),
        ],
        out_specs=[
            pl.BlockSpec((blk, D), lambda i: (i, 0)),
            pl.BlockSpec((blk, 2 * D), lambda i: (i, 0)),
        ],
        out_shape=[
            jax.ShapeDtypeStruct((N, D), jnp.float32),
            jax.ShapeDtypeStruct((N, 2 * D), jnp.float32),
        ],
    )(x, Wqkv, S, aux)


def _edge_proj_body(ea_ref, we_ref, e_ref):
    e_ref[...] = jnp.dot(ea_ref[...], we_ref[...],
                         preferred_element_type=jnp.float32)


def _edge_proj(edge_attr, We):
    blk = 4000
    return pl.pallas_call(
        _edge_proj_body,
        grid=(E // blk,),
        in_specs=[
            pl.BlockSpec((blk, D), lambda i: (i, 0)),
            pl.BlockSpec((D, D), lambda i: (0, 0)),
        ],
        out_specs=pl.BlockSpec((blk, D), lambda i: (i, 0)),
        out_shape=jax.ShapeDtypeStruct((E, D), jnp.float32),
    )(edge_attr, We)


def _edge_pass_body(src_hbm, dst_hbm, dstd_hbm, q_hbm, kv_hbm, e_hbm,
                    ziota_hbm, num_hbm, den_hbm,
                    srcv, dstv, dstp, dst16, dstd16, idzv, qv, kvv, ev, cv,
                    av, zbuf, acc_sh, sem_q, sem_kv, sem_e):
    cid = lax.axis_index("c")
    sid = lax.axis_index("s")
    wid = sid * NC + cid
    lane = lax.iota(jnp.int32, 16)
    zero16 = jnp.zeros((16,), jnp.float32)

    def zfill(i, carry):
        for h in range(H):
            zbuf[i, pl.ds(h * 16, 16)] = zero16
        return carry

    lax.fori_loop(0, ZGRP, zfill, 0)

    def zscat(j, carry):
        off = sid * ZROWS_PER_SUB + j * ZGRP
        pltpu.sync_copy(ziota_hbm.at[pl.ds(off, ZGRP)], idzv)
        pltpu.sync_copy(zbuf, acc_sh.at[idzv])
        return carry

    lax.fori_loop(0, ZROWS_PER_SUB // ZGRP, zscat, 0)
    plsc.subcore_barrier()

    def chunk(jc, carry):
        base = wid * EPW + jc * EB
        pltpu.sync_copy(src_hbm.at[pl.ds(base, EB)], srcv)
        pltpu.sync_copy(dst_hbm.at[pl.ds(base, EB)], dstv)
        pltpu.sync_copy(dst_hbm.at[pl.ds(base, EB)], dstp.at[pl.ds(0, EB)])
        cp_q = pltpu.async_copy(q_hbm.at[dstv], qv, sem_q)
        cp_kv = pltpu.async_copy(kv_hbm.at[srcv], kvv, sem_kv)
        cp_e = pltpu.async_copy(e_hbm.at[pl.ds(base, EB)], ev, sem_e)
        cp_q.wait()
        cp_kv.wait()
        cp_e.wait()

        def edge(i, carry2):
            arow = zero16
            for h in range(H):
                qh = qv[i, pl.ds(h * 16, 16)]
                kh = kvv[i, pl.ds(h * 16, 16)]
                vh = kvv[i, pl.ds(D + h * 16, 16)]
                eh = ev[i, pl.ds(h * 16, 16)]
                t = qh * (kh + eh)
                sh = jnp.sum(t) * QK_SCALE
                a_h = jnp.exp(jnp.full((16,), sh, jnp.float32))
                cv[i, pl.ds(h * 16, 16)] = (vh + eh) * a_h
                arow = jnp.where(lane == h, a_h, arow)
            slot = dstp[pl.ds(i, 16)][0] & 7
            for sl in range(8):
                av[i, pl.ds(sl * 16, 16)] = jnp.where(slot == sl, arow,
                                                      zero16)
            return carry2

        lax.fori_loop(0, EB, edge, 0)

        def group(jg, carry3):
            gbase = base + jg * GB
            pltpu.sync_copy(dst_hbm.at[pl.ds(gbase, GB)], dst16)
            pltpu.sync_copy(dstd_hbm.at[pl.ds(gbase, GB)], dstd16)
            pltpu.sync_copy(cv.at[pl.ds(jg * GB, GB)], acc_sh.at[dst16],
                            add=True)
            pltpu.sync_copy(av.at[pl.ds(jg * GB, GB)], acc_sh.at[dstd16],
                            add=True)
            return carry3

        lax.fori_loop(0, NG, group, 0)
        return carry

    lax.fori_loop(0, NCHUNK, chunk, 0)
    plsc.subcore_barrier()

    for j in range(NROWS_PER_SUB // DRAIN_ROWS):
        off = sid * NROWS_PER_SUB + j * DRAIN_ROWS
        pltpu.sync_copy(acc_sh.at[pl.ds(off, DRAIN_ROWS)],
                        num_hbm.at[cid, pl.ds(off, DRAIN_ROWS)])
    doff = sid * DROWS_PER_SUB
    pltpu.sync_copy(acc_sh.at[pl.ds(NPAD + doff, DROWS_PER_SUB)],
                    den_hbm.at[cid, pl.ds(doff, DROWS_PER_SUB)])


def _edge_pass(src, dst, dstd, q, kv, e, ziota):
    mesh = plsc.VectorSubcoreMesh(core_axis_name="c", subcore_axis_name="s")
    f = pl.kernel(
        _edge_pass_body,
        out_type=(
            jax.ShapeDtypeStruct((NC, NPAD, D), jnp.float32),
            jax.ShapeDtypeStruct((NC, DROWS, D), jnp.float32),
        ),
        mesh=mesh,
        scratch_types=[
            pltpu.VMEM((EB,), jnp.int32),
            pltpu.VMEM((EB,), jnp.int32),
            pltpu.VMEM((EB + 16,), jnp.int32),
            pltpu.VMEM((GB,), jnp.int32),
            pltpu.VMEM((GB,), jnp.int32),
            pltpu.VMEM((ZGRP,), jnp.int32),
            pltpu.VMEM((EB, D), jnp.float32),
            pltpu.VMEM((EB, 2 * D), jnp.float32),
            pltpu.VMEM((EB, D), jnp.float32),
            pltpu.VMEM((EB, D), jnp.float32),
            pltpu.VMEM((EB, D), jnp.float32),
            pltpu.VMEM((ZGRP, D), jnp.float32),
            pltpu.VMEM_SHARED((TROWS, D), jnp.float32),
            pltpu.SemaphoreType.DMA,
            pltpu.SemaphoreType.DMA,
            pltpu.SemaphoreType.DMA,
        ],
        compiler_params=pltpu.CompilerParams(needs_layout_passes=False),
    )
    return f(src, dst, dstd, q, kv, e, ziota)


def _out_body(num_ref, den_ref, b_ref, wo_ref, o_ref):
    nm = num_ref[0] + num_ref[1]
    dn = den_ref[0] + den_ref[1]
    dbc = jnp.dot(dn, b_ref[...], preferred_element_type=jnp.float32)
    r = nm / jnp.where(dbc > 0, dbc, 1.0)
    o_ref[...] = jnp.dot(r, wo_ref[...], preferred_element_type=jnp.float32)


def _out_proj(num, den, Bm, Wo):
    blk = 640
    return pl.pallas_call(
        _out_body,
        grid=(NPAD // blk,),
        in_specs=[
            pl.BlockSpec((2, blk, D), lambda i: (0, i, 0)),
            pl.BlockSpec((2, blk, 16), lambda i: (0, i, 0)),
            pl.BlockSpec((16, D), lambda i: (0, 0)),
            pl.BlockSpec((D, D), lambda i: (0, 0)),
        ],
        out_specs=pl.BlockSpec((blk, D), lambda i: (i, 0)),
        out_shape=jax.ShapeDtypeStruct((NPAD, D), jnp.float32),
    )(num, den, Bm, Wo)


def _norm_consts(w_q_norm, w_k_norm):
    S = np.zeros((3 * D, 3 * D), np.float32)
    for g in range(2 * D // C):
        S[g * C:(g + 1) * C, g * C:(g + 1) * C] = 1.0 / C
    epsvec = np.concatenate([np.full((2 * D,), 1e-6, np.float32),
                             np.ones((D,), np.float32)])
    wvec = jnp.concatenate([jnp.tile(w_q_norm, H), jnp.tile(w_k_norm, H),
                            jnp.ones((D,), jnp.float32)])
    aux = jnp.stack([jnp.asarray(epsvec), wvec])
    Bm = np.zeros((16, D), np.float32)
    for h in range(H):
        Bm[h, h * C:(h + 1) * C] = 1.0
    return jnp.asarray(S), aux, jnp.asarray(Bm)


def kernel(x, edge_index, edge_attr, Wq, Wk, Wv, We, Wo, w_q_norm, w_k_norm):
    Wqkv = jnp.concatenate([Wq, Wk, Wv], axis=1)
    S, aux, Bm = _norm_consts(w_q_norm, w_k_norm)
    q, kv = _qkv(x, Wqkv, S, aux)
    e = _edge_proj(edge_attr, We)
    src = edge_index[0]
    dst = edge_index[1]
    dstd = NPAD + lax.shift_right_logical(dst, 3)
    ziota = jnp.arange(TROWS, dtype=jnp.int32)
    num, den_packed = _edge_pass(src, dst, dstd, q, kv, e, ziota)
    den = den_packed.reshape(NC, NPAD, 16)
    return _out_proj(num, den, Bm, Wo)[:N]


# speedup vs baseline: 12.2324x; 1.3278x over previous
import functools

import jax
import jax.numpy as jnp
import numpy as np
from jax import lax
from jax.experimental import pallas as pl
from jax.experimental.pallas import tpu as pltpu
from jax.experimental.pallas import tpu_sc as plsc

N = 10000
E = 320000
D = 128
H = 8
C = 16

NC = 2
NS = 16
NW = NC * NS
EPW = E // NW
EB = 40
GB = 8
ZGRP = 16
NG = EB // GB
NCHUNK = EPW // EB
NPAD = 10240
DROWS = NPAD // 8
TROWS = NPAD + DROWS
NROWS_PER_SUB = NPAD // NS
DROWS_PER_SUB = DROWS // NS
ZROWS_PER_SUB = TROWS // NS
DRAIN_ROWS = 128

QK_SCALE = 1.0 / np.sqrt(float(C))


def _qkv_body(x_ref, w_ref, s_ref, aux_ref, q_ref, kv_ref):
    y = jnp.dot(x_ref[...], w_ref[...], preferred_element_type=jnp.float32)
    g = jnp.dot(y * y, s_ref[...], preferred_element_type=jnp.float32)
    z = y * lax.rsqrt(g + aux_ref[0]) * aux_ref[1]
    q_ref[...] = z[:, :D]
    kv_ref[...] = z[:, D:]


def _qkv(x, Wqkv, S, aux):
    blk = 1000
    return pl.pallas_call(
        _qkv_body,
        grid=(N // blk,),
        in_specs=[
            pl.BlockSpec((blk, D), lambda i: (i, 0)),
            pl.BlockSpec((D, 3 * D), lambda i: (0, 0)),
            pl.BlockSpec((3 * D, 3 * D), lambda i: (0, 0)),
            pl.BlockSpec((2, 3 * D), lambda i: (0, 0)),
        ],
        out_specs=[
            pl.BlockSpec((blk, D), lambda i: (i, 0)),
            pl.BlockSpec((blk, 2 * D), lambda i: (i, 0)),
        ],
        out_shape=[
            jax.ShapeDtypeStruct((N, D), jnp.float32),
            jax.ShapeDtypeStruct((N, 2 * D), jnp.float32),
        ],
    )(x, Wqkv, S, aux)


def _edge_proj_body(ea_ref, we_ref, e_ref):
    e_ref[...] = jnp.dot(ea_ref[...], we_ref[...],
                         preferred_element_type=jnp.float32)


def _edge_proj(edge_attr, We):
    blk = 4000
    return pl.pallas_call(
        _edge_proj_body,
        grid=(E // blk,),
        in_specs=[
            pl.BlockSpec((blk, D), lambda i: (i, 0)),
            pl.BlockSpec((D, D), lambda i: (0, 0)),
        ],
        out_specs=pl.BlockSpec((blk, D), lambda i: (i, 0)),
        out_shape=jax.ShapeDtypeStruct((E, D), jnp.float32),
    )(edge_attr, We)


def _edge_pass_body(src_hbm, dst_hbm, dstcat_hbm, q_hbm, kv_hbm, e_hbm,
                    ziota_hbm, num_hbm, den_hbm,
                    sv0, dv0, dcg0, idzv, qv0, kvv, ev0, cav, zbuf, acc_sh,
                    sem_i0, sem_g0, sem_kv, sem_sc):
    cid = lax.axis_index("c")
    sid = lax.axis_index("s")
    wid = sid * NC + cid
    lane = lax.iota(jnp.int32, 16)
    zero16 = jnp.zeros((16,), jnp.float32)
    svs = (sv0,)
    dvs = (dv0,)
    dcgs = (dcg0,)
    qvs = (qv0,)
    evs = (ev0,)
    sem_i = (sem_i0,)
    sem_g = (sem_g0,)

    def zfill(i, carry):
        for h in range(H):
            zbuf[i, pl.ds(h * 16, 16)] = zero16
        return carry

    lax.fori_loop(0, ZGRP, zfill, 0)

    def zscat(j, carry):
        off = sid * ZROWS_PER_SUB + j * ZGRP
        pltpu.sync_copy(ziota_hbm.at[pl.ds(off, ZGRP)], idzv)
        pltpu.sync_copy(zbuf, acc_sh.at[idzv])
        return carry

    lax.fori_loop(0, ZROWS_PER_SUB // ZGRP, zscat, 0)
    plsc.subcore_barrier()

    def _rbase(jc):
        return wid * (EPW // GB) + jc * NG

    def issue_idx(jc, b):
        jcc = jnp.minimum(jc, NCHUNK - 1)
        base = wid * EPW + jcc * EB
        ra = _rbase(jcc)
        ra_al = pl.multiple_of(ra - (ra & 7), 8)
        pltpu.async_copy(src_hbm.at[pl.ds(base, EB)], svs[b], sem_i[b])
        pltpu.async_copy(dst_hbm.at[pl.ds(base, EB)],
                         dvs[b].at[pl.ds(0, EB)], sem_i[b])
        pltpu.async_copy(dstcat_hbm.at[pl.ds(ra_al, 16)], dcgs[b], sem_i[b])

    def wait_idx(b):
        pltpu.make_async_copy(src_hbm.at[pl.ds(0, EB)], svs[b],
                              sem_i[b]).wait()
        pltpu.make_async_copy(dst_hbm.at[pl.ds(0, EB)],
                              dvs[b].at[pl.ds(0, EB)], sem_i[b]).wait()
        pltpu.make_async_copy(dstcat_hbm.at[pl.ds(0, 16)], dcgs[b],
                              sem_i[b]).wait()

    def issue_gathers(jc, b):
        jcc = jnp.minimum(jc, NCHUNK - 1)
        base = wid * EPW + jcc * EB
        pltpu.async_copy(q_hbm.at[dvs[b].at[pl.ds(0, EB)]], qvs[b],
                         sem_g[b])
        pltpu.async_copy(e_hbm.at[pl.ds(base, EB)], evs[b], sem_g[b])

    def wait_gathers(b):
        pltpu.make_async_copy(q_hbm.at[pl.ds(0, EB)], qvs[b],
                              sem_g[b]).wait()
        pltpu.make_async_copy(e_hbm.at[pl.ds(0, EB)], evs[b],
                              sem_g[b]).wait()

    def outer(jc, carry):
        for b in (0,):
            issue_idx(jc, b)
            wait_idx(b)
            cp_kv = pltpu.async_copy(kv_hbm.at[svs[b]], kvv, sem_kv)
            issue_gathers(jc, b)
            cp_kv.wait()
            wait_gathers(b)

            def edge(i, carry2):
                r_cv = ((i >> 3) << 4) + (i & 7)
                arow = zero16
                for h in range(H):
                    qh = qvs[b][i, pl.ds(h * 16, 16)]
                    kh = kvv[i, pl.ds(h * 16, 16)]
                    vh = kvv[i, pl.ds(D + h * 16, 16)]
                    eh = evs[b][i, pl.ds(h * 16, 16)]
                    t = qh * (kh + eh)
                    sh = jnp.sum(t) * QK_SCALE
                    a_h = jnp.exp(jnp.full((16,), sh, jnp.float32))
                    cav[r_cv, pl.ds(h * 16, 16)] = (vh + eh) * a_h
                    arow = jnp.where(lane == h, a_h, arow)
                slot = dvs[b][pl.ds(i, 16)][0] & 7
                for sl in range(8):
                    cav[r_cv + GB, pl.ds(sl * 16, 16)] = jnp.where(
                        slot == sl, arow, zero16)
                return carry2

            lax.fori_loop(0, EB, edge, 0)
            roff = _rbase(jnp.minimum(jc, NCHUNK - 1)) & 7
            scs = [
                pltpu.async_copy(cav.at[pl.ds(jg * 2 * GB, 2 * GB)],
                                 acc_sh.at[dcgs[b].at[jg + roff]], sem_sc,
                                 add=True)
                for jg in range(NG)
            ]
            for d in scs:
                d.wait()
        return carry

    lax.fori_loop(0, NCHUNK, outer, 0)
    plsc.subcore_barrier()

    def drain_num(j, carry):
        off = sid * NROWS_PER_SUB + j * ZGRP
        pltpu.sync_copy(acc_sh.at[pl.ds(off, ZGRP)],
                        num_hbm.at[cid, pl.ds(off, ZGRP)])
        return carry

    lax.fori_loop(0, NROWS_PER_SUB // ZGRP, drain_num, 0)

    def drain_den(j, carry):
        doff = sid * DROWS_PER_SUB + j * ZGRP
        pltpu.sync_copy(acc_sh.at[pl.ds(NPAD + doff, ZGRP)],
                        den_hbm.at[cid, pl.ds(doff, ZGRP)])
        return carry

    lax.fori_loop(0, DROWS_PER_SUB // ZGRP, drain_den, 0)


def _edge_pass(src, dst, dstcat, q, kv, e, ziota):
    mesh = plsc.VectorSubcoreMesh(core_axis_name="c", subcore_axis_name="s")
    f = pl.kernel(
        _edge_pass_body,
        out_type=(
            jax.ShapeDtypeStruct((NC, NPAD, D), jnp.float32),
            jax.ShapeDtypeStruct((NC, DROWS, D), jnp.float32),
        ),
        mesh=mesh,
        scratch_types=[
            pltpu.VMEM((EB,), jnp.int32),
            pltpu.VMEM((EB + 16,), jnp.int32),
            pltpu.VMEM((16, 2 * GB), jnp.int32),
            pltpu.VMEM((ZGRP,), jnp.int32),
            pltpu.VMEM((EB, D), jnp.float32),
            pltpu.VMEM((EB, 2 * D), jnp.float32),
            pltpu.VMEM((EB, D), jnp.float32),
            pltpu.VMEM((2 * EB, D), jnp.float32),
            pltpu.VMEM((ZGRP, D), jnp.float32),
            pltpu.VMEM_SHARED((TROWS, D), jnp.float32),
            pltpu.SemaphoreType.DMA,
            pltpu.SemaphoreType.DMA,
            pltpu.SemaphoreType.DMA,
            pltpu.SemaphoreType.DMA,
        ],
        compiler_params=pltpu.CompilerParams(needs_layout_passes=False),
    )
    return f(src, dst, dstcat, q, kv, e, ziota)


def _out_body(num_ref, den_ref, b_ref, wo_ref, o_ref):
    nm = num_ref[0] + num_ref[1]
    dn = den_ref[0] + den_ref[1]
    dbc = jnp.dot(dn, b_ref[...], preferred_element_type=jnp.float32)
    r = nm / jnp.where(dbc > 0, dbc, 1.0)
    o_ref[...] = jnp.dot(r, wo_ref[...], preferred_element_type=jnp.float32)


def _out_proj(num, den, Bm, Wo):
    blk = 640
    return pl.pallas_call(
        _out_body,
        grid=(NPAD // blk,),
        in_specs=[
            pl.BlockSpec((2, blk, D), lambda i: (0, i, 0)),
            pl.BlockSpec((2, blk, 16), lambda i: (0, i, 0)),
            pl.BlockSpec((16, D), lambda i: (0, 0)),
            pl.BlockSpec((D, D), lambda i: (0, 0)),
        ],
        out_specs=pl.BlockSpec((blk, D), lambda i: (i, 0)),
        out_shape=jax.ShapeDtypeStruct((NPAD, D), jnp.float32),
    )(num, den, Bm, Wo)


def _norm_consts(w_q_norm, w_k_norm):
    S = np.zeros((3 * D, 3 * D), np.float32)
    for g in range(2 * D // C):
        S[g * C:(g + 1) * C, g * C:(g + 1) * C] = 1.0 / C
    epsvec = np.concatenate([np.full((2 * D,), 1e-6, np.float32),
                             np.ones((D,), np.float32)])
    wvec = jnp.concatenate([jnp.tile(w_q_norm, H), jnp.tile(w_k_norm, H),
                            jnp.ones((D,), jnp.float32)])
    aux = jnp.stack([jnp.asarray(epsvec), wvec])
    Bm = np.zeros((16, D), np.float32)
    for h in range(H):
        Bm[h, h * C:(h + 1) * C] = 1.0
    return jnp.asarray(S), aux, jnp.asarray(Bm)


def kernel(x, edge_index, edge_attr, Wq, Wk, Wv, We, Wo, w_q_norm, w_k_norm):
    Wqkv = jnp.concatenate([Wq, Wk, Wv], axis=1)
    S, aux, Bm = _norm_consts(w_q_norm, w_k_norm)
    q, kv = _qkv(x, Wqkv, S, aux)
    e = _edge_proj(edge_attr, We)
    src = edge_index[0]
    dst = edge_index[1]
    dstd = NPAD + lax.shift_right_logical(dst, 3)
    dstcat = jnp.concatenate([dst.reshape(-1, GB), dstd.reshape(-1, GB)],
                             axis=1)
    ziota = jnp.arange(TROWS, dtype=jnp.int32)
    num, den_packed = _edge_pass(src, dst, dstcat, q, kv, e, ziota)
    den = den_packed.reshape(NC, NPAD, 16)
    return _out_proj(num, den, Bm, Wo)[:N]


# speedup vs baseline: 12.7143x; 1.0394x over previous
import functools

import jax
import jax.numpy as jnp
import numpy as np
from jax import lax
from jax.experimental import pallas as pl
from jax.experimental.pallas import tpu as pltpu
from jax.experimental.pallas import tpu_sc as plsc

N = 10000
E = 320000
D = 128
H = 8
C = 16

NC = 2
NS = 16
NW = NC * NS
EPW = E // NW
EB = 40
GB = 8
ZGRP = 16
NG = EB // GB
NCHUNK = EPW // EB
NPAD = 10240
DROWS = NPAD // 8
TROWS = NPAD + DROWS
NROWS_PER_SUB = NPAD // NS
DROWS_PER_SUB = DROWS // NS
ZROWS_PER_SUB = TROWS // NS
DRAIN_ROWS = 128

QK_SCALE = 1.0 / np.sqrt(float(C))


def _qkv_body(x_ref, w_ref, s_ref, aux_ref, q_ref, kv_ref):
    y = jnp.dot(x_ref[...], w_ref[...], preferred_element_type=jnp.float32)
    g = jnp.dot(y * y, s_ref[...], preferred_element_type=jnp.float32)
    z = y * lax.rsqrt(g + aux_ref[0]) * aux_ref[1]
    q_ref[...] = z[:, :D]
    kv_ref[...] = z[:, D:]


def _qkv(x, Wqkv, S, aux):
    blk = 1000
    return pl.pallas_call(
        _qkv_body,
        grid=(N // blk,),
        in_specs=[
            pl.BlockSpec((blk, D), lambda i: (i, 0)),
            pl.BlockSpec((D, 3 * D), lambda i: (0, 0)),
            pl.BlockSpec((3 * D, 3 * D), lambda i: (0, 0)),
            pl.BlockSpec((2, 3 * D), lambda i: (0, 0)),
        ],
        out_specs=[
            pl.BlockSpec((blk, D), lambda i: (i, 0)),
            pl.BlockSpec((blk, 2 * D), lambda i: (i, 0)),
        ],
        out_shape=[
            jax.ShapeDtypeStruct((N, D), jnp.float32),
            jax.ShapeDtypeStruct((N, 2 * D), jnp.float32),
        ],
    )(x, Wqkv, S, aux)


def _edge_proj_body(ea_ref, we_ref, e_ref):
    e_ref[...] = jnp.dot(ea_ref[...], we_ref[...],
                         preferred_element_type=jnp.float32)


def _edge_proj(edge_attr, We):
    blk = 4000
    return pl.pallas_call(
        _edge_proj_body,
        grid=(E // blk,),
        in_specs=[
            pl.BlockSpec((blk, D), lambda i: (i, 0)),
            pl.BlockSpec((D, D), lambda i: (0, 0)),
        ],
        out_specs=pl.BlockSpec((blk, D), lambda i: (i, 0)),
        out_shape=jax.ShapeDtypeStruct((E, D), jnp.float32),
    )(edge_attr, We)


def _edge_pass_body(src_hbm, dst_hbm, dstcat_hbm, q_hbm, kv_hbm, e_hbm,
                    ziota_hbm, num_hbm, den_hbm,
                    sv0, sv1, dv0, dv1, dcg0, dcg1, idzv, qv0, kvv, ev0,
                    cav, zbuf, acc_sh,
                    sem_i0, sem_i1, sem_g0, sem_kv, sem_sc):
    cid = lax.axis_index("c")
    sid = lax.axis_index("s")
    wid = sid * NC + cid
    lane = lax.iota(jnp.int32, 16)
    zero16 = jnp.zeros((16,), jnp.float32)
    svs = (sv0, sv1)
    dvs = (dv0, dv1)
    dcgs = (dcg0, dcg1)
    qvs = (qv0, qv0)
    evs = (ev0, ev0)
    sem_i = (sem_i0, sem_i1)
    sem_g = (sem_g0, sem_g0)

    def zfill(i, carry):
        for h in range(H):
            zbuf[i, pl.ds(h * 16, 16)] = zero16
        return carry

    lax.fori_loop(0, ZGRP, zfill, 0)

    def zscat(j, carry):
        off = sid * ZROWS_PER_SUB + j * ZGRP
        pltpu.sync_copy(ziota_hbm.at[pl.ds(off, ZGRP)], idzv)
        pltpu.sync_copy(zbuf, acc_sh.at[idzv])
        return carry

    lax.fori_loop(0, ZROWS_PER_SUB // ZGRP, zscat, 0)
    plsc.subcore_barrier()

    def _rbase(jc):
        return wid * (EPW // GB) + jc * NG

    def issue_idx(jc, b):
        jcc = jnp.minimum(jc, NCHUNK - 1)
        base = wid * EPW + jcc * EB
        ra = _rbase(jcc)
        ra_al = pl.multiple_of(ra - (ra & 7), 8)
        pltpu.async_copy(src_hbm.at[pl.ds(base, EB)], svs[b], sem_i[b])
        pltpu.async_copy(dst_hbm.at[pl.ds(base, EB)],
                         dvs[b].at[pl.ds(0, EB)], sem_i[b])
        pltpu.async_copy(dstcat_hbm.at[pl.ds(ra_al, 16)], dcgs[b], sem_i[b])

    def wait_idx(b):
        pltpu.make_async_copy(src_hbm.at[pl.ds(0, EB)], svs[b],
                              sem_i[b]).wait()
        pltpu.make_async_copy(dst_hbm.at[pl.ds(0, EB)],
                              dvs[b].at[pl.ds(0, EB)], sem_i[b]).wait()
        pltpu.make_async_copy(dstcat_hbm.at[pl.ds(0, 16)], dcgs[b],
                              sem_i[b]).wait()

    def issue_gathers(jc, b):
        jcc = jnp.minimum(jc, NCHUNK - 1)
        base = wid * EPW + jcc * EB
        pltpu.async_copy(q_hbm.at[dvs[b].at[pl.ds(0, EB)]], qvs[b],
                         sem_g[b])
        pltpu.async_copy(e_hbm.at[pl.ds(base, EB)], evs[b], sem_g[b])

    def wait_gathers(b):
        pltpu.make_async_copy(q_hbm.at[pl.ds(0, EB)], qvs[b],
                              sem_g[b]).wait()
        pltpu.make_async_copy(e_hbm.at[pl.ds(0, EB)], evs[b],
                              sem_g[b]).wait()

    issue_idx(0, 0)

    def outer(jo, carry):
        for b in (0, 1):
            jc = 2 * jo + b
            nb = 1 - b
            wait_idx(b)
            issue_idx(jc + 1, nb)
            cp_kv = pltpu.async_copy(kv_hbm.at[svs[b]], kvv, sem_kv)
            issue_gathers(jc, b)
            cp_kv.wait()
            wait_gathers(b)

            def edge(i, carry2):
                r_cv = ((i >> 3) << 4) + (i & 7)
                arow = zero16
                for h in range(H):
                    qh = qvs[b][i, pl.ds(h * 16, 16)]
                    kh = kvv[i, pl.ds(h * 16, 16)]
                    vh = kvv[i, pl.ds(D + h * 16, 16)]
                    eh = evs[b][i, pl.ds(h * 16, 16)]
                    t = qh * (kh + eh)
                    sh = jnp.sum(t) * QK_SCALE
                    a_h = jnp.exp(jnp.full((16,), sh, jnp.float32))
                    cav[r_cv, pl.ds(h * 16, 16)] = (vh + eh) * a_h
                    arow = jnp.where(lane == h, a_h, arow)
                slot = dvs[b][pl.ds(i, 16)][0] & 7
                for sl in range(8):
                    cav[r_cv + GB, pl.ds(sl * 16, 16)] = jnp.where(
                        slot == sl, arow, zero16)
                return carry2

            lax.fori_loop(0, EB, edge, 0)
            roff = _rbase(jnp.minimum(jc, NCHUNK - 1)) & 7
            scs = [
                pltpu.async_copy(cav.at[pl.ds(jg * 2 * GB, 2 * GB)],
                                 acc_sh.at[dcgs[b].at[jg + roff]], sem_sc,
                                 add=True)
                for jg in range(NG)
            ]
            for d in scs:
                d.wait()
        return carry

    lax.fori_loop(0, NCHUNK // 2, outer, 0)
    wait_idx(0)
    plsc.subcore_barrier()

    def drain_num(j, carry):
        off = sid * NROWS_PER_SUB + j * ZGRP
        pltpu.sync_copy(acc_sh.at[pl.ds(off, ZGRP)],
                        num_hbm.at[cid, pl.ds(off, ZGRP)])
        return carry

    lax.fori_loop(0, NROWS_PER_SUB // ZGRP, drain_num, 0)

    def drain_den(j, carry):
        doff = sid * DROWS_PER_SUB + j * ZGRP
        pltpu.sync_copy(acc_sh.at[pl.ds(NPAD + doff, ZGRP)],
                        den_hbm.at[cid, pl.ds(doff, ZGRP)])
        return carry

    lax.fori_loop(0, DROWS_PER_SUB // ZGRP, drain_den, 0)


def _edge_pass(src, dst, dstcat, q, kv, e, ziota):
    mesh = plsc.VectorSubcoreMesh(core_axis_name="c", subcore_axis_name="s")
    f = pl.kernel(
        _edge_pass_body,
        out_type=(
            jax.ShapeDtypeStruct((NC, NPAD, D), jnp.float32),
            jax.ShapeDtypeStruct((NC, DROWS, D), jnp.float32),
        ),
        mesh=mesh,
        scratch_types=[
            pltpu.VMEM((EB,), jnp.int32),
            pltpu.VMEM((EB,), jnp.int32),
            pltpu.VMEM((EB + 16,), jnp.int32),
            pltpu.VMEM((EB + 16,), jnp.int32),
            pltpu.VMEM((16, 2 * GB), jnp.int32),
            pltpu.VMEM((16, 2 * GB), jnp.int32),
            pltpu.VMEM((ZGRP,), jnp.int32),
            pltpu.VMEM((EB, D), jnp.float32),
            pltpu.VMEM((EB, 2 * D), jnp.float32),
            pltpu.VMEM((EB, D), jnp.float32),
            pltpu.VMEM((2 * EB, D), jnp.float32),
            pltpu.VMEM((ZGRP, D), jnp.float32),
            pltpu.VMEM_SHARED((TROWS, D), jnp.float32),
            pltpu.SemaphoreType.DMA,
            pltpu.SemaphoreType.DMA,
            pltpu.SemaphoreType.DMA,
            pltpu.SemaphoreType.DMA,
            pltpu.SemaphoreType.DMA,
        ],
        compiler_params=pltpu.CompilerParams(needs_layout_passes=False),
    )
    return f(src, dst, dstcat, q, kv, e, ziota)


def _out_body(num_ref, den_ref, b_ref, wo_ref, o_ref):
    nm = num_ref[0] + num_ref[1]
    dn = den_ref[0] + den_ref[1]
    dbc = jnp.dot(dn, b_ref[...], preferred_element_type=jnp.float32)
    r = nm / jnp.where(dbc > 0, dbc, 1.0)
    o_ref[...] = jnp.dot(r, wo_ref[...], preferred_element_type=jnp.float32)


def _out_proj(num, den, Bm, Wo):
    blk = 640
    return pl.pallas_call(
        _out_body,
        grid=(NPAD // blk,),
        in_specs=[
            pl.BlockSpec((2, blk, D), lambda i: (0, i, 0)),
            pl.BlockSpec((2, blk, 16), lambda i: (0, i, 0)),
            pl.BlockSpec((16, D), lambda i: (0, 0)),
            pl.BlockSpec((D, D), lambda i: (0, 0)),
        ],
        out_specs=pl.BlockSpec((blk, D), lambda i: (i, 0)),
        out_shape=jax.ShapeDtypeStruct((NPAD, D), jnp.float32),
    )(num, den, Bm, Wo)


def _norm_consts(w_q_norm, w_k_norm):
    S = np.zeros((3 * D, 3 * D), np.float32)
    for g in range(2 * D // C):
        S[g * C:(g + 1) * C, g * C:(g + 1) * C] = 1.0 / C
    epsvec = np.concatenate([np.full((2 * D,), 1e-6, np.float32),
                             np.ones((D,), np.float32)])
    wvec = jnp.concatenate([jnp.tile(w_q_norm, H), jnp.tile(w_k_norm, H),
                            jnp.ones((D,), jnp.float32)])
    aux = jnp.stack([jnp.asarray(epsvec), wvec])
    Bm = np.zeros((16, D), np.float32)
    for h in range(H):
        Bm[h, h * C:(h + 1) * C] = 1.0
    return jnp.asarray(S), aux, jnp.asarray(Bm)


def kernel(x, edge_index, edge_attr, Wq, Wk, Wv, We, Wo, w_q_norm, w_k_norm):
    Wqkv = jnp.concatenate([Wq, Wk, Wv], axis=1)
    S, aux, Bm = _norm_consts(w_q_norm, w_k_norm)
    q, kv = _qkv(x, Wqkv, S, aux)
    e = _edge_proj(edge_attr, We)
    src = edge_index[0]
    dst = edge_index[1]
    dstd = NPAD + lax.shift_right_logical(dst, 3)
    dstcat = jnp.concatenate([dst.reshape(-1, GB), dstd.reshape(-1, GB)],
                             axis=1)
    ziota = jnp.arange(TROWS, dtype=jnp.int32)
    num, den_packed = _edge_pass(src, dst, dstcat, q, kv, e, ziota)
    den = den_packed.reshape(NC, NPAD, 16)
    return _out_proj(num, den, Bm, Wo)[:N]
